# baseline (speedup 1.0000x reference)
import functools

import jax
import jax.numpy as jnp
from jax import lax
from jax.experimental import pallas as pl
from jax.experimental.pallas import tpu as pltpu
from jax.experimental.pallas import tpu_sc as plsc

VOCAB = 400000
EMBED_DIM = 300
PAD_DIM = 384
BATCH = 16384

_info = plsc.get_sparse_core_info()
_NC, _NS = _info.num_cores, _info.num_subcores
_NW = _NC * _NS
_B_PER_W = BATCH // _NW
_CHUNK = 64
_NCHUNK = _B_PER_W // _CHUNK

_TR_BV = 6400


def _transpose_pad_body(t_ref, out_ref):
    x = t_ref[...]
    x = jnp.pad(x, ((0, PAD_DIM - EMBED_DIM), (0, 0)))
    out_ref[...] = x.T


def _transpose_pad(table_t):
    return pl.pallas_call(
        _transpose_pad_body,
        out_shape=jax.ShapeDtypeStruct((VOCAB, PAD_DIM), jnp.float32),
        grid=(-(-VOCAB // _TR_BV),),
        in_specs=[pl.BlockSpec((EMBED_DIM, _TR_BV), lambda i: (0, i))],
        out_specs=pl.BlockSpec((_TR_BV, PAD_DIM), lambda i: (i, 0)),
    )(table_t)


def _gather_kernel(tbl_hbm, idx_hbm, out_hbm, idx_v, sG, outb, gsem, osem):
    wid = lax.axis_index("s") * _NC + lax.axis_index("c")
    pltpu.sync_copy(idx_hbm.at[wid], idx_v)

    def fire(c, buf):
        pltpu.async_copy(tbl_hbm.at[idx_v.at[c]], sG.at[buf], gsem)

    def wait_gather(buf):
        pltpu.make_async_copy(tbl_hbm.at[idx_v.at[0]], sG.at[buf], gsem).wait()

    def out_copy(c, buf):
        rowbase = wid * _B_PER_W + c * _CHUNK
        return pltpu.make_async_copy(
            outb.at[buf], out_hbm.at[pl.ds(rowbase, _CHUNK)], osem)

    def assemble(buf):
        for r in range(_CHUNK):
            for t in range(18):
                outb[buf, r, pl.ds(16 * t, 16)] = sG[buf, r, pl.ds(16 * t, 16)]
            outb[buf, r, pl.ds(284, 16)] = sG[buf, r, pl.ds(284, 16)]

    fire(0, 0)

    def do_pair(p, carry):
        c0 = 2 * p
        c1 = c0 + 1

        wait_gather(0)
        fire(c1, 1)

        @pl.when(p >= 1)
        def _():
            out_copy(c0 - 2, 0).wait()

        assemble(0)
        out_copy(c0, 0).start()

        wait_gather(1)

        @pl.when(p + 1 < _NCHUNK // 2)
        def _():
            fire(c1 + 1, 0)

        @pl.when(p >= 1)
        def _():
            out_copy(c1 - 2, 1).wait()

        assemble(1)
        out_copy(c1, 1).start()
        return carry

    lax.fori_loop(0, _NCHUNK // 2, do_pair, 0)
    out_copy(_NCHUNK - 2, 0).wait()
    out_copy(_NCHUNK - 1, 1).wait()


@jax.jit
def _glove_gather(captions, glove_vectors):
    tbl = _transpose_pad(glove_vectors.T)
    idx = captions.reshape(_NW, _NCHUNK, _CHUNK)

    k = functools.partial(
        pl.kernel,
        out_type=jax.ShapeDtypeStruct((BATCH, EMBED_DIM), jnp.float32),
        mesh=plsc.VectorSubcoreMesh(core_axis_name="c", subcore_axis_name="s"),
        scratch_types=[
            pltpu.VMEM((_NCHUNK, _CHUNK), jnp.int32),
            pltpu.VMEM((2, _CHUNK, PAD_DIM), jnp.float32),
            pltpu.VMEM((2, _CHUNK, EMBED_DIM), jnp.float32),
            pltpu.SemaphoreType.DMA,
            pltpu.SemaphoreType.DMA,
        ],
        compiler_params=pltpu.CompilerParams(
            use_tc_tiling_on_sc=True, needs_layout_passes=False
        ),
    )(_gather_kernel)
    return k(tbl, idx)


def kernel(class_labels, captions, glove_vectors):
    return _glove_gather(captions, glove_vectors)


# speedup vs baseline: 1.0023x; 1.0023x over previous
import functools

import jax
import jax.numpy as jnp
from jax import lax
from jax.experimental import pallas as pl
from jax.experimental.pallas import tpu as pltpu
from jax.experimental.pallas import tpu_sc as plsc

VOCAB = 400000
EMBED_DIM = 300
PAD_DIM = 384
BATCH = 16384

_info = plsc.get_sparse_core_info()
_NC, _NS = _info.num_cores, _info.num_subcores
_NW = _NC * _NS
_B_PER_W = BATCH // _NW
_CHUNK = 32
_NCHUNK = _B_PER_W // _CHUNK

_TR_BV = 6400


def _transpose_pad_body(t_ref, out_ref):
    x = t_ref[...]
    x = jnp.pad(x, ((0, PAD_DIM - EMBED_DIM), (0, 0)))
    out_ref[...] = x.T


def _transpose_pad(table_t):
    return pl.pallas_call(
        _transpose_pad_body,
        out_shape=jax.ShapeDtypeStruct((VOCAB, PAD_DIM), jnp.float32),
        grid=(-(-VOCAB // _TR_BV),),
        in_specs=[pl.BlockSpec((EMBED_DIM, _TR_BV), lambda i: (0, i))],
        out_specs=pl.BlockSpec((_TR_BV, PAD_DIM), lambda i: (i, 0)),
    )(table_t)


def _gather_kernel(tbl_hbm, idx_hbm, out_hbm, idx_v, sG, gsem, osem):
    wid = lax.axis_index("s") * _NC + lax.axis_index("c")
    pltpu.sync_copy(idx_hbm.at[wid], idx_v)

    def fire(c, buf):
        pltpu.async_copy(tbl_hbm.at[idx_v.at[c]], sG.at[buf], gsem)

    def wait_gather(buf):
        pltpu.make_async_copy(tbl_hbm.at[idx_v.at[0]], sG.at[buf], gsem).wait()

    def out_copy(c, buf):
        rowbase = wid * _B_PER_W + c * _CHUNK
        return pltpu.make_async_copy(
            sG.at[buf], out_hbm.at[pl.ds(rowbase, _CHUNK)], osem)

    fire(0, 0)

    def do_pair(p, carry):
        c0 = 2 * p
        c1 = c0 + 1

        wait_gather(0)
        fire(c1, 1)

        @pl.when(p >= 1)
        def _():
            out_copy(c0 - 2, 0).wait()

        out_copy(c0, 0).start()

        wait_gather(1)

        @pl.when(p + 1 < _NCHUNK // 2)
        def _():
            fire(c1 + 1, 0)

        @pl.when(p >= 1)
        def _():
            out_copy(c1 - 2, 1).wait()

        out_copy(c1, 1).start()
        return carry

    lax.fori_loop(0, _NCHUNK // 2, do_pair, 0)
    out_copy(_NCHUNK - 2, 0).wait()
    out_copy(_NCHUNK - 1, 1).wait()


@jax.jit
def _glove_gather(captions, glove_vectors):
    tbl = _transpose_pad(glove_vectors.T)
    idx = captions.reshape(_NW, _NCHUNK, _CHUNK)

    k = functools.partial(
        pl.kernel,
        out_type=jax.ShapeDtypeStruct((BATCH, PAD_DIM), jnp.float32),
        mesh=plsc.VectorSubcoreMesh(core_axis_name="c", subcore_axis_name="s"),
        scratch_types=[
            pltpu.VMEM((_NCHUNK, _CHUNK), jnp.int32),
            pltpu.VMEM((2, _CHUNK, PAD_DIM), jnp.float32),
            pltpu.SemaphoreType.DMA,
            pltpu.SemaphoreType.DMA,
        ],
        compiler_params=pltpu.CompilerParams(
            use_tc_tiling_on_sc=True, needs_layout_passes=False
        ),
    )(_gather_kernel)
    out = k(tbl, idx)
    return lax.slice(out, (0, 0), (BATCH, EMBED_DIM))


def kernel(class_labels, captions, glove_vectors):
    return _glove_gather(captions, glove_vectors)
